# baseline (speedup 1.0000x reference)
import functools

import numpy as np

import jax
import jax.numpy as jnp
from jax import lax
from jax.experimental import pallas as pl
from jax.experimental.pallas import tpu as pltpu
from jax.experimental.pallas import tpu_sc as plsc

_NC, _NS = 2, 16
_NW = _NC * _NS
_B, _NEG, _D = 1024, 200, 128
_NT = _NEG + 1
_CH = 128
_NSUB = _NT * _B // _CH
_SPW = _NSUB // _NW
_XTRA = _NSUB - _SPW * _NW
_NB = 4
_SPAD = 52
_BPC = _B // _CH
_HPW = _B // _NW


def _sc_gather(entity, relation, head_idx, rel_idx, tail_idx):
    mesh = plsc.VectorSubcoreMesh(core_axis_name="c", subcore_axis_name="s")

    @functools.partial(
        pl.kernel,
        mesh=mesh,
        out_type=[
            jax.ShapeDtypeStruct((_B, _D), jnp.float32),
            jax.ShapeDtypeStruct((_B, _D), jnp.float32),
            jax.ShapeDtypeStruct((_NT, _B, _D), jnp.float32),
        ],
        scratch_types=[
            pltpu.VMEM((_HPW,), jnp.int32),
            pltpu.VMEM((_HPW, _D), jnp.float32),
            pltpu.VMEM((_SPAD, _CH), jnp.int32),
        ] + [pltpu.VMEM((_CH, _D), jnp.float32) for _ in range(_NB)]
          + [pltpu.SemaphoreType.DMA for _ in range(2 * _NB)],
    )
    def k(ent_hbm, rel_hbm, hidx_hbm, ridx_hbm, tidx_hbm,
          head_out, rel_out, tail_out,
          sidx_v, srow_v, tidx_v, *bufs_and_sems):
        bufs = bufs_and_sems[:_NB]
        gsem = bufs_and_sems[_NB:2 * _NB]
        ssem = bufs_and_sems[2 * _NB:]
        wid = lax.axis_index("s") * _NC + lax.axis_index("c")

        hbase = wid * _HPW
        pltpu.sync_copy(hidx_hbm.at[wid], sidx_v)
        pltpu.async_copy(ent_hbm.at[sidx_v], srow_v, gsem[0]).wait()
        pltpu.sync_copy(srow_v, head_out.at[pl.ds(hbase, _HPW)])

        pltpu.sync_copy(ridx_hbm.at[wid], sidx_v)
        pltpu.async_copy(rel_hbm.at[sidx_v], srow_v, gsem[0]).wait()
        pltpu.sync_copy(srow_v, rel_out.at[pl.ds(hbase, _HPW)])

        start = _SPW * wid + jnp.minimum(wid, _XTRA)
        cnt = _SPW + jnp.where(wid < _XTRA, 1, 0)
        pltpu.sync_copy(tidx_hbm.at[wid], tidx_v)

        def out_slice(j):
            t = start + j
            n = t // _BPC
            off = (t % _BPC) * _CH
            return tail_out.at[n, pl.ds(off, _CH)]

        def gather_start(j, b):
            pltpu.async_copy(ent_hbm.at[tidx_v.at[j]], bufs[b], gsem[b])

        def gather_wait(b):
            pltpu.make_async_copy(
                ent_hbm.at[tidx_v.at[0]], bufs[b], gsem[b]).wait()

        for b in range(_NB):
            gather_start(b, b)

        def body(g, carry):
            for b in range(_NB):
                j = g * _NB + b
                gather_wait(b)
                pltpu.async_copy(bufs[b], out_slice(j), ssem[b])
            for b in range(_NB):
                pltpu.make_async_copy(bufs[b], out_slice(0), ssem[b]).wait()
                gather_start((g + 1) * _NB + b, b)
            return carry

        lax.fori_loop(0, _SPAD // _NB - 1, body, 0)

        last = _SPAD - _NB
        for b in range(_NB):
            j = last + b
            gather_wait(b)

            @pl.when(j < cnt)
            def _():
                pltpu.sync_copy(bufs[b], out_slice(j))

    return k(entity, relation, head_idx, rel_idx, tail_idx)


def kernel(positive, negative, entity_embedding, relation_embedding):
    positive = positive.astype(jnp.int32)
    negative = negative.astype(jnp.int32)
    head_idx = positive[:, 0].reshape(_NW, _HPW)
    rel_idx = positive[:, 1].reshape(_NW, _HPW)
    tail_idx = jnp.concatenate([positive[:, 2:3], negative], axis=1)
    flat = jnp.pad(tail_idx.T.reshape(_NSUB, _CH), ((0, _SPAD, ), (0, 0)))
    starts = np.minimum(np.arange(_NW), _XTRA) + _SPW * np.arange(_NW)
    rows = starts[:, None] + np.arange(_SPAD)[None, :]
    tail_idx = flat[rows]
    head, rel, tail = _sc_gather(
        entity_embedding, relation_embedding, head_idx, rel_idx, tail_idx)
    return (head[:, None, :], rel[:, None, :], tail.transpose(1, 0, 2))


# speedup vs baseline: 1.1910x; 1.1910x over previous
import functools

import jax
import jax.numpy as jnp
from jax import lax
from jax.experimental import pallas as pl
from jax.experimental.pallas import tpu as pltpu
from jax.experimental.pallas import tpu_sc as plsc

_NC, _NS = 2, 16
_NW = _NC * _NS
_B, _NEG, _D = 1024, 200, 128
_NT = _NEG + 1
_CH = 128
_NSUB = _NT * _B // _CH
_SPW = _NSUB // _NW
_XTRA = _NSUB - _SPW * _NW
_SPAD = _SPW + 1
_BPC = _B // _CH
_HPW = _B // _NW


def _sc_gather(entity, relation, head_idx, rel_idx, tail_idx):
    mesh = plsc.VectorSubcoreMesh(core_axis_name="c", subcore_axis_name="s")

    @functools.partial(
        pl.kernel,
        mesh=mesh,
        out_type=[
            jax.ShapeDtypeStruct((_B, _D), jnp.float32),
            jax.ShapeDtypeStruct((_B, _D), jnp.float32),
            jax.ShapeDtypeStruct((_NT, _B, _D), jnp.float32),
        ],
        scratch_types=[
            pltpu.VMEM((_HPW,), jnp.int32),
            pltpu.VMEM((_HPW, _D), jnp.float32),
            pltpu.VMEM((_SPAD * _CH,), jnp.int32),
            pltpu.VMEM((_CH, _D), jnp.float32),
            pltpu.VMEM((_CH, _D), jnp.float32),
            pltpu.SemaphoreType.DMA,
            pltpu.SemaphoreType.DMA,
        ],
    )
    def k(ent_hbm, rel_hbm, hidx_hbm, ridx_hbm, tidx_hbm,
          head_out, rel_out, tail_out,
          sidx_v, srow_v, tidx_v, buf0, buf1, sem0, sem1):
        bufs = (buf0, buf1)
        sems = (sem0, sem1)
        wid = lax.axis_index("s") * _NC + lax.axis_index("c")

        hbase = wid * _HPW
        pltpu.sync_copy(hidx_hbm.at[pl.ds(hbase, _HPW)], sidx_v)
        pltpu.async_copy(ent_hbm.at[sidx_v], srow_v, sem0).wait()
        pltpu.sync_copy(srow_v, head_out.at[pl.ds(hbase, _HPW)])

        pltpu.sync_copy(ridx_hbm.at[pl.ds(hbase, _HPW)], sidx_v)
        pltpu.async_copy(rel_hbm.at[sidx_v], srow_v, sem0).wait()
        pltpu.sync_copy(srow_v, rel_out.at[pl.ds(hbase, _HPW)])

        start = _SPW * wid + jnp.minimum(wid, _XTRA)
        pltpu.sync_copy(
            tidx_hbm.at[pl.ds(start * _CH, _SPAD * _CH)], tidx_v)

        def gather_start(j, b):
            pltpu.async_copy(
                ent_hbm.at[tidx_v.at[pl.ds(j * _CH, _CH)]], bufs[b], sems[b])

        def gather_wait(b):
            pltpu.make_async_copy(
                ent_hbm.at[tidx_v.at[pl.ds(0, _CH)]], bufs[b], sems[b]).wait()

        def store(j, b):
            t = start + j
            n = t // _BPC
            off = (t % _BPC) * _CH
            pltpu.sync_copy(bufs[b], tail_out.at[n, pl.ds(off, _CH)])

        gather_start(0, 0)

        def body(i, carry):
            j0 = 2 * i
            gather_start(j0 + 1, 1)
            gather_wait(0)
            store(j0, 0)
            gather_start(j0 + 2, 0)
            gather_wait(1)
            store(j0 + 1, 1)
            return carry

        lax.fori_loop(0, _SPW // 2 - 1, body, 0)

        gather_start(_SPW - 1, 1)
        gather_wait(0)
        store(_SPW - 2, 0)
        gather_wait(1)
        store(_SPW - 1, 1)

        @pl.when(wid < _XTRA)
        def _():
            gather_start(_SPW, 0)
            gather_wait(0)
            store(_SPW, 0)

    return k(entity, relation, head_idx, rel_idx, tail_idx)


def kernel(positive, negative, entity_embedding, relation_embedding):
    positive = positive.astype(jnp.int32)
    negative = negative.astype(jnp.int32)
    head_idx = positive[:, 0]
    rel_idx = positive[:, 1]
    tail_idx = jnp.concatenate([positive[:, 2:3], negative], axis=1)
    tail_idx = jnp.pad(tail_idx.T.reshape(-1), (0, _SPAD * _CH))
    head, rel, tail = _sc_gather(
        entity_embedding, relation_embedding, head_idx, rel_idx, tail_idx)
    return (head[:, None, :], rel[:, None, :], tail.transpose(1, 0, 2))


# speedup vs baseline: 1.2131x; 1.0186x over previous
import functools

import jax
import jax.numpy as jnp
from jax import lax
from jax.experimental import pallas as pl
from jax.experimental.pallas import tpu as pltpu
from jax.experimental.pallas import tpu_sc as plsc

_NC, _NS = 2, 16
_NW = _NC * _NS
_B, _NEG, _D = 1024, 200, 128
_NT = _NEG + 1
_CH = 128
_NSUB = _NT * _B // _CH
_SPW = _NSUB // _NW
_XTRA = _NSUB - _SPW * _NW
_SPAD = _SPW + 1
_BPC = _B // _CH
_HPW = _B // _NW


def _sc_gather(entity, relation, head_idx, rel_idx, tail_idx):
    mesh = plsc.VectorSubcoreMesh(core_axis_name="c", subcore_axis_name="s")

    @functools.partial(
        pl.kernel,
        mesh=mesh,
        out_type=[
            jax.ShapeDtypeStruct((_B, _D), jnp.float32),
            jax.ShapeDtypeStruct((_B, _D), jnp.float32),
            jax.ShapeDtypeStruct((_NT, _B, _D), jnp.float32),
        ],
        scratch_types=[
            pltpu.VMEM((_HPW,), jnp.int32),
            pltpu.VMEM((_HPW, _D), jnp.float32),
            pltpu.VMEM((_SPAD * _CH,), jnp.int32),
        ] + [pltpu.VMEM((_CH, _D), jnp.float32) for _ in range(4)]
          + [pltpu.SemaphoreType.DMA for _ in range(8)],
    )
    def k(ent_hbm, rel_hbm, hidx_hbm, ridx_hbm, tidx_hbm,
          head_out, rel_out, tail_out,
          sidx_v, srow_v, tidx_v, *rest):
        bufs = rest[:4]
        sems = rest[4:8]
        ssem = rest[8:]
        sem0 = sems[0]
        wid = lax.axis_index("s") * _NC + lax.axis_index("c")

        hbase = wid * _HPW
        pltpu.sync_copy(hidx_hbm.at[pl.ds(hbase, _HPW)], sidx_v)
        pltpu.async_copy(ent_hbm.at[sidx_v], srow_v, sem0).wait()
        pltpu.sync_copy(srow_v, head_out.at[pl.ds(hbase, _HPW)])

        pltpu.sync_copy(ridx_hbm.at[pl.ds(hbase, _HPW)], sidx_v)
        pltpu.async_copy(rel_hbm.at[sidx_v], srow_v, sem0).wait()
        pltpu.sync_copy(srow_v, rel_out.at[pl.ds(hbase, _HPW)])

        start = _SPW * wid + jnp.minimum(wid, _XTRA)
        pltpu.sync_copy(
            tidx_hbm.at[pl.ds(start * _CH, _SPAD * _CH)], tidx_v)

        def gather_start(j, b):
            pltpu.async_copy(
                ent_hbm.at[tidx_v.at[pl.ds(j * _CH, _CH)]], bufs[b], sems[b])

        def gather_wait(b):
            pltpu.make_async_copy(
                ent_hbm.at[tidx_v.at[pl.ds(0, _CH)]], bufs[b], sems[b]).wait()

        def out_slice(j):
            t = start + j
            n = t // _BPC
            off = (t % _BPC) * _CH
            return tail_out.at[n, pl.ds(off, _CH)]

        def store_start(j, b):
            pltpu.async_copy(bufs[b], out_slice(j), ssem[b])

        def store_wait(b):
            pltpu.make_async_copy(bufs[b], out_slice(0), ssem[b]).wait()

        gather_start(0, 0)
        gather_start(1, 1)

        gather_wait(0)
        store_start(0, 0)
        gather_start(2, 2)

        gather_wait(1)
        store_start(1, 1)
        gather_start(3, 3)

        def body(i, carry):
            for u in range(4):
                j = 4 * i + 2 + u
                b = (2 + u) % 4
                gather_wait(b)
                store_start(j, b)
                store_wait((u + 4) % 4)
                gather_start(j + 2, (u + 4) % 4)
            return carry

        lax.fori_loop(0, (_SPW - 6) // 4, body, 0)

        for u in range(2):
            j = _SPW - 4 + u
            b = j % 4
            gather_wait(b)
            store_start(j, b)
            store_wait((j + 2) % 4)
            gather_start(j + 2, (j + 2) % 4)
        for u in range(2):
            j = _SPW - 2 + u
            b = j % 4
            gather_wait(b)
            store_start(j, b)
        for u in range(4):
            store_wait((_SPW - 4 + u) % 4)

        @pl.when(wid < _XTRA)
        def _():
            gather_start(_SPW, 0)
            gather_wait(0)
            pltpu.sync_copy(bufs[0], out_slice(_SPW))

    return k(entity, relation, head_idx, rel_idx, tail_idx)


def kernel(positive, negative, entity_embedding, relation_embedding):
    positive = positive.astype(jnp.int32)
    negative = negative.astype(jnp.int32)
    head_idx = positive[:, 0]
    rel_idx = positive[:, 1]
    tail_idx = jnp.concatenate([positive[:, 2:3], negative], axis=1)
    tail_idx = jnp.pad(tail_idx.T.reshape(-1), (0, _SPAD * _CH))
    head, rel, tail = _sc_gather(
        entity_embedding, relation_embedding, head_idx, rel_idx, tail_idx)
    return (head[:, None, :], rel[:, None, :], tail.transpose(1, 0, 2))


# speedup vs baseline: 1.2310x; 1.0147x over previous
import functools

import jax
import jax.numpy as jnp
from jax import lax
from jax.experimental import pallas as pl
from jax.experimental.pallas import tpu as pltpu
from jax.experimental.pallas import tpu_sc as plsc

_NC, _NS = 2, 16
_NW = _NC * _NS
_B, _NEG, _D = 1024, 200, 128
_NT = _NEG + 1
_CH = 128
_NSUB = _NT * _B // _CH
_SPW = _NSUB // _NW
_XTRA = _NSUB - _SPW * _NW
_SPAD = _SPW + 1
_BPC = _B // _CH
_HPW = _B // _NW


def _sc_gather(entity, relation, head_idx, rel_idx, tail_idx):
    mesh = plsc.VectorSubcoreMesh(core_axis_name="c", subcore_axis_name="s")

    @functools.partial(
        pl.kernel,
        mesh=mesh,
        out_type=[
            jax.ShapeDtypeStruct((_B, _D), jnp.float32),
            jax.ShapeDtypeStruct((_B, _D), jnp.float32),
            jax.ShapeDtypeStruct((_NT, _B, _D), jnp.float32),
        ],
        scratch_types=[
            pltpu.VMEM((_HPW,), jnp.int32),
            pltpu.VMEM((_HPW,), jnp.int32),
            pltpu.VMEM((_HPW, _D), jnp.float32),
            pltpu.VMEM((_HPW, _D), jnp.float32),
            pltpu.VMEM((_SPAD * _CH,), jnp.int32),
        ] + [pltpu.VMEM((_CH, _D), jnp.float32) for _ in range(4)]
          + [pltpu.SemaphoreType.DMA for _ in range(11)],
    )
    def k(ent_hbm, rel_hbm, hidx_hbm, ridx_hbm, tidx_hbm,
          head_out, rel_out, tail_out,
          sidx_v, sidx2_v, srow_v, srow2_v, tidx_v, *rest):
        bufs = rest[:4]
        sems = rest[4:8]
        ssem = rest[8:12]
        isem, hsem, rsem = rest[12:]
        wid = lax.axis_index("s") * _NC + lax.axis_index("c")

        start = _SPW * wid + jnp.minimum(wid, _XTRA)
        pltpu.async_copy(
            tidx_hbm.at[pl.ds(start * _CH, _SPAD * _CH)], tidx_v, isem)

        hbase = wid * _HPW
        pltpu.sync_copy(hidx_hbm.at[pl.ds(hbase, _HPW)], sidx_v)
        pltpu.async_copy(ent_hbm.at[sidx_v], srow_v, hsem)
        pltpu.sync_copy(ridx_hbm.at[pl.ds(hbase, _HPW)], sidx2_v)
        pltpu.async_copy(rel_hbm.at[sidx2_v], srow2_v, rsem)

        pltpu.make_async_copy(
            tidx_hbm.at[pl.ds(0, _SPAD * _CH)], tidx_v, isem).wait()

        def gather_start(j, b):
            pltpu.async_copy(
                ent_hbm.at[tidx_v.at[pl.ds(j * _CH, _CH)]], bufs[b], sems[b])

        def gather_wait(b):
            pltpu.make_async_copy(
                ent_hbm.at[tidx_v.at[pl.ds(0, _CH)]], bufs[b], sems[b]).wait()

        def out_slice(j):
            t = start + j
            n = t // _BPC
            off = (t % _BPC) * _CH
            return tail_out.at[n, pl.ds(off, _CH)]

        def store_start(j, b):
            pltpu.async_copy(bufs[b], out_slice(j), ssem[b])

        def store_wait(b):
            pltpu.make_async_copy(bufs[b], out_slice(0), ssem[b]).wait()

        gather_start(0, 0)
        gather_start(1, 1)

        gather_wait(0)
        store_start(0, 0)
        gather_start(2, 2)

        gather_wait(1)
        store_start(1, 1)
        gather_start(3, 3)

        def body(i, carry):
            for u in range(4):
                j = 4 * i + 2 + u
                b = (2 + u) % 4
                gather_wait(b)
                store_start(j, b)
                store_wait((u + 4) % 4)
                gather_start(j + 2, (u + 4) % 4)
            return carry

        lax.fori_loop(0, (_SPW - 6) // 4, body, 0)

        for u in range(2):
            j = _SPW - 4 + u
            b = j % 4
            gather_wait(b)
            store_start(j, b)
            store_wait((j + 2) % 4)
            gather_start(j + 2, (j + 2) % 4)
        for u in range(2):
            j = _SPW - 2 + u
            b = j % 4
            gather_wait(b)
            store_start(j, b)
        for u in range(4):
            store_wait((_SPW - 4 + u) % 4)

        @pl.when(wid < _XTRA)
        def _():
            gather_start(_SPW, 0)
            gather_wait(0)
            pltpu.sync_copy(bufs[0], out_slice(_SPW))

        pltpu.make_async_copy(ent_hbm.at[sidx_v], srow_v, hsem).wait()
        pltpu.sync_copy(srow_v, head_out.at[pl.ds(hbase, _HPW)])
        pltpu.make_async_copy(rel_hbm.at[sidx2_v], srow2_v, rsem).wait()
        pltpu.sync_copy(srow2_v, rel_out.at[pl.ds(hbase, _HPW)])

    return k(entity, relation, head_idx, rel_idx, tail_idx)


def kernel(positive, negative, entity_embedding, relation_embedding):
    positive = positive.astype(jnp.int32)
    negative = negative.astype(jnp.int32)
    head_idx = positive[:, 0]
    rel_idx = positive[:, 1]
    tail_idx = jnp.concatenate([positive[:, 2:3], negative], axis=1)
    tail_idx = jnp.pad(tail_idx.T.reshape(-1), (0, _SPAD * _CH))
    head, rel, tail = _sc_gather(
        entity_embedding, relation_embedding, head_idx, rel_idx, tail_idx)
    return (head[:, None, :], rel[:, None, :], tail.transpose(1, 0, 2))


# speedup vs baseline: 1.2405x; 1.0078x over previous
import functools

import jax
import jax.numpy as jnp
from jax import lax
from jax.experimental import pallas as pl
from jax.experimental.pallas import tpu as pltpu
from jax.experimental.pallas import tpu_sc as plsc

_NC, _NS = 2, 16
_NW = _NC * _NS
_B, _NEG, _D = 1024, 200, 128
_NT = _NEG + 1
_CH = 128
_NSUB = _NT * _B // _CH
_SPW = _NSUB // _NW
_XTRA = _NSUB - _SPW * _NW
_SPAD = _SPW + 1
_BPC = _B // _CH
_HPW = _B // _NW


def _sc_gather(entity, relation, head_idx, rel_idx, tail_idx):
    mesh = plsc.VectorSubcoreMesh(core_axis_name="c", subcore_axis_name="s")

    @functools.partial(
        pl.kernel,
        mesh=mesh,
        out_type=[
            jax.ShapeDtypeStruct((_B, _D), jnp.float32),
            jax.ShapeDtypeStruct((_B, _D), jnp.float32),
            jax.ShapeDtypeStruct((_NT, _B, _D), jnp.float32),
        ],
        scratch_types=[
            pltpu.VMEM((_HPW,), jnp.int32),
            pltpu.VMEM((_HPW,), jnp.int32),
            pltpu.VMEM((_HPW, _D), jnp.float32),
            pltpu.VMEM((_HPW, _D), jnp.float32),
            pltpu.VMEM((_SPAD * _CH,), jnp.int32),
        ] + [pltpu.VMEM((_CH, _D), jnp.float32) for _ in range(6)]
          + [pltpu.SemaphoreType.DMA for _ in range(15)],
    )
    def k(ent_hbm, rel_hbm, hidx_hbm, ridx_hbm, tidx_hbm,
          head_out, rel_out, tail_out,
          sidx_v, sidx2_v, srow_v, srow2_v, tidx_v, *rest):
        bufs = rest[:6]
        sems = rest[6:12]
        ssem = rest[12:18]
        isem, hsem, rsem = rest[18:]
        wid = lax.axis_index("s") * _NC + lax.axis_index("c")

        start = _SPW * wid + jnp.minimum(wid, _XTRA)
        pltpu.async_copy(
            tidx_hbm.at[pl.ds(start * _CH, _SPAD * _CH)], tidx_v, isem)

        hbase = wid * _HPW
        pltpu.sync_copy(hidx_hbm.at[pl.ds(hbase, _HPW)], sidx_v)
        pltpu.async_copy(ent_hbm.at[sidx_v], srow_v, hsem)
        pltpu.sync_copy(ridx_hbm.at[pl.ds(hbase, _HPW)], sidx2_v)
        pltpu.async_copy(rel_hbm.at[sidx2_v], srow2_v, rsem)

        pltpu.make_async_copy(
            tidx_hbm.at[pl.ds(0, _SPAD * _CH)], tidx_v, isem).wait()

        def gather_start(j, b):
            pltpu.async_copy(
                ent_hbm.at[tidx_v.at[pl.ds(j * _CH, _CH)]], bufs[b], sems[b])

        def gather_wait(b):
            pltpu.make_async_copy(
                ent_hbm.at[tidx_v.at[pl.ds(0, _CH)]], bufs[b], sems[b]).wait()

        def out_slice(j):
            t = start + j
            n = t // _BPC
            off = (t % _BPC) * _CH
            return tail_out.at[n, pl.ds(off, _CH)]

        def store_start(j, b):
            pltpu.async_copy(bufs[b], out_slice(j), ssem[b])

        def store_wait(b):
            pltpu.make_async_copy(bufs[b], out_slice(0), ssem[b]).wait()

        for j in range(3):
            gather_start(j, j)
        for j in range(3):
            gather_wait(j)
            store_start(j, j)
            gather_start(j + 3, j + 3)

        def body(i, carry):
            for u in range(6):
                j = 6 * i + 3 + u
                b = (3 + u) % 6
                gather_wait(b)
                store_start(j, b)
                store_wait((u + 6) % 6)
                gather_start(j + 3, (u + 6) % 6)
            return carry

        lax.fori_loop(0, (_SPW - 8) // 6, body, 0)

        for u in range(2):
            j = _SPW - 5 + u
            b = j % 6
            gather_wait(b)
            store_start(j, b)
            store_wait((j + 3) % 6)
            gather_start(j + 3, (j + 3) % 6)
        for u in range(3):
            j = _SPW - 3 + u
            b = j % 6
            gather_wait(b)
            store_start(j, b)
        for u in range(6):
            store_wait((_SPW - 6 + u) % 6)

        @pl.when(wid < _XTRA)
        def _():
            gather_start(_SPW, 0)
            gather_wait(0)
            pltpu.sync_copy(bufs[0], out_slice(_SPW))

        pltpu.make_async_copy(ent_hbm.at[sidx_v], srow_v, hsem).wait()
        pltpu.sync_copy(srow_v, head_out.at[pl.ds(hbase, _HPW)])
        pltpu.make_async_copy(rel_hbm.at[sidx2_v], srow2_v, rsem).wait()
        pltpu.sync_copy(srow2_v, rel_out.at[pl.ds(hbase, _HPW)])

    return k(entity, relation, head_idx, rel_idx, tail_idx)


def kernel(positive, negative, entity_embedding, relation_embedding):
    positive = positive.astype(jnp.int32)
    negative = negative.astype(jnp.int32)
    head_idx = positive[:, 0]
    rel_idx = positive[:, 1]
    tail_idx = jnp.concatenate([positive[:, 2:3], negative], axis=1)
    tail_idx = jnp.pad(tail_idx.T.reshape(-1), (0, _SPAD * _CH))
    head, rel, tail = _sc_gather(
        entity_embedding, relation_embedding, head_idx, rel_idx, tail_idx)
    return (head[:, None, :], rel[:, None, :], tail.transpose(1, 0, 2))
